# baseline (speedup 1.0000x reference)
import functools

import jax
import jax.numpy as jnp
from jax.experimental import pallas as pl
from jax.experimental.pallas import tpu as pltpu

_N = 10000
_T = 21
_F_IN = 128
_HID = 32
_BN = 400
_SCALE = 1.0 / (1.0 + 1e-5) ** 0.5


def _pack_taps(p):
    w1, b1, w2, b2, w3, b3 = p
    taps = [
        jnp.concatenate([w1[:, :, 0, k].T, w2[:, :, 0, k].T, w3[:, :, 0, k].T], axis=1)
        for k in range(3)
    ]
    b = jnp.concatenate([b1, b2, b3]).reshape(1, 3 * _HID)
    return taps, b


def _pack_stacked(p):
    taps, b = _pack_taps(p)
    return jnp.concatenate(taps, axis=0), b


def _glu(Y):
    P = Y[:, :, 0:32]
    Q = Y[:, :, 32:64]
    R = Y[:, :, 64:96]
    return jax.nn.relu(P * jax.nn.sigmoid(Q) + R)


def _tap_cat(H, t_out):
    return jnp.concatenate(
        [H[0:t_out], H[1:t_out + 1], H[2:t_out + 2]], axis=2)


def _mm(x3d, w):
    t, bn, c = x3d.shape
    y = jnp.dot(x3d.reshape(t * bn, c), w, preferred_element_type=jnp.float32)
    return y.reshape(t, bn, w.shape[1])


def _stgcn_block(x_ref, w1k0_ref, w1k1_ref, w1k2_ref, b1_ref, wa_ref, ba_ref,
                 w2_ref, b2_ref, w3_ref, b3_ref, wb_ref, bb_ref, w4_ref,
                 b4_ref, out_ref):
    xb = x_ref[...]
    A0 = _mm(xb, w1k0_ref[...])
    A1 = _mm(xb, w1k1_ref[...])
    A2 = _mm(xb, w1k2_ref[...])
    Y1 = A0[0:19] + A1[1:20] + A2[2:21] + b1_ref[...][None]
    H1 = _glu(Y1)
    Tc = jax.nn.relu(_mm(H1, wa_ref[...]) + ba_ref[...][None])
    H2 = _glu(_mm(_tap_cat(Tc, 17), w2_ref[...]) + b2_ref[...][None]) * _SCALE
    H3 = _glu(_mm(_tap_cat(H2, 15), w3_ref[...]) + b3_ref[...][None])
    Tc2 = jax.nn.relu(_mm(H3, wb_ref[...]) + bb_ref[...][None])
    H4 = _glu(_mm(_tap_cat(Tc2, 13), w4_ref[...]) + b4_ref[...][None])

    out_ref[0] = jnp.sum(H4, axis=1)


def _finish_block(part_ref, lw_ref, lb_ref, out_ref):
    acc = jnp.sum(part_ref[...], axis=0)
    s = jnp.sum(acc, axis=1, keepdims=True)
    out = jnp.sum(s * lw_ref[...], axis=0, keepdims=True)
    out_ref[...] = out * (_SCALE / (_N * _HID)) + lb_ref[...]


def kernel(x, edge_index, edge_weight, tc1a, cheb_a, tc2a, tc1b, cheb_b, tc2b,
           lin_w, lin_b):
    del edge_index, edge_weight
    (W1k0, W1k1, W1k2), B1 = _pack_taps(tc1a)
    W2, B2 = _pack_stacked(tc2a)
    W3, B3 = _pack_stacked(tc1b)
    W4, B4 = _pack_stacked(tc2b)
    Wa, ba = cheb_a
    Wb, bb = cheb_b
    ba = ba.reshape(1, _HID)
    bb = bb.reshape(1, _HID)
    lb = lin_b.reshape(1, -1)

    nblocks = _N // _BN
    full = lambda a: pl.BlockSpec(a.shape, lambda *_: tuple(0 for _ in a.shape))
    parts = pl.pallas_call(
        _stgcn_block,
        grid=(nblocks,),
        in_specs=[
            pl.BlockSpec((_T, _BN, _F_IN), lambda i: (0, i, 0)),
            full(W1k0), full(W1k1), full(W1k2), full(B1), full(Wa), full(ba),
            full(W2), full(B2), full(W3), full(B3), full(Wb), full(bb),
            full(W4), full(B4),
        ],
        out_specs=pl.BlockSpec((1, 13, _HID), lambda i: (i, 0, 0)),
        out_shape=jax.ShapeDtypeStruct((nblocks, 13, _HID), jnp.float32),
        compiler_params=pltpu.CompilerParams(
            dimension_semantics=("parallel",)),
    )(x, W1k0, W1k1, W1k2, B1, Wa, ba, W2, B2, W3, B3, Wb, bb, W4, B4)
    out = pl.pallas_call(
        _finish_block,
        in_specs=[full(parts), full(lin_w), full(lb)],
        out_specs=pl.BlockSpec((1, lin_w.shape[1]), lambda *_: (0, 0)),
        out_shape=jax.ShapeDtypeStruct((1, lin_w.shape[1]), jnp.float32),
    )(parts, lin_w, lb)
    return out[0]


# speedup vs baseline: 3.3744x; 3.3744x over previous
import functools

import jax
import jax.numpy as jnp
from jax.experimental import pallas as pl
from jax.experimental.pallas import tpu as pltpu

_N = 10000
_T = 21
_F_IN = 128
_HID = 32
_BN = 384
_SCALE = 1.0 / (1.0 + 1e-5) ** 0.5


def _pack_taps_t(p):
    w1, b1, w2, b2, w3, b3 = p
    taps = [
        jnp.concatenate([w1[:, :, 0, k], w2[:, :, 0, k], w3[:, :, 0, k]], axis=0)
        for k in range(3)
    ]
    b = jnp.concatenate([b1, b2, b3]).reshape(3 * _HID, 1)
    return taps, b


def _pack_stacked_t(p):
    taps, b = _pack_taps_t(p)
    return jnp.concatenate(taps, axis=1), b


def _glu_t(Y):
    P = Y[0:32, :]
    Q = Y[32:64, :]
    R = Y[64:96, :]
    return jax.nn.relu(P * jax.nn.sigmoid(Q) + R)


def _tap_stack(H, t_out):
    L = t_out * _BN
    return jnp.concatenate(
        [H[:, 0:L], H[:, _BN:_BN + L], H[:, 2 * _BN:2 * _BN + L]], axis=0)


def _stgcn_block(x_ref, mask_ref, w1k0_ref, w1k1_ref, w1k2_ref, b1_ref,
                 wa_ref, ba_ref, w2_ref, b2_ref, w3_ref, b3_ref, wb_ref,
                 bb_ref, w4_ref, b4_ref, lw_ref, lb_ref, out_ref, acc_ref,
                 *, nblocks):
    i = pl.program_id(0)

    xb = x_ref[...]
    X3 = jnp.transpose(xb, (0, 2, 1))
    xT = jnp.concatenate([X3[t] for t in range(_T)], axis=1)

    dot = functools.partial(jnp.dot, preferred_element_type=jnp.float32)
    A0 = dot(w1k0_ref[...], xT)
    A1 = dot(w1k1_ref[...], xT)
    A2 = dot(w1k2_ref[...], xT)
    L1 = 19 * _BN
    Y1 = (A0[:, 0:L1] + A1[:, _BN:_BN + L1] + A2[:, 2 * _BN:2 * _BN + L1]
          + b1_ref[...])
    H1 = _glu_t(Y1)
    Tc = jax.nn.relu(dot(wa_ref[...], H1) + ba_ref[...])
    H2 = _glu_t(dot(w2_ref[...], _tap_stack(Tc, 17)) + b2_ref[...]) * _SCALE
    H3 = _glu_t(dot(w3_ref[...], _tap_stack(H2, 15)) + b3_ref[...])
    Tc2 = jax.nn.relu(dot(wb_ref[...], H3) + bb_ref[...])
    H4 = _glu_t(dot(w4_ref[...], _tap_stack(Tc2, 13)) + b4_ref[...])

    mask = jnp.concatenate([mask_ref[0]] * 13, axis=1)
    H4 = jnp.where(mask > 0, H4, 0.0)
    part = jnp.sum(H4, axis=0, keepdims=True)

    @pl.when(i == 0)
    def _init():
        acc_ref[...] = jnp.zeros_like(acc_ref)

    acc_ref[...] += part

    @pl.when(i == nblocks - 1)
    def _finish():
        acc = acc_ref[...]
        a13 = jnp.concatenate(
            [acc[:, t * _BN:(t + 1) * _BN] for t in range(13)], axis=0)
        s = jnp.sum(a13, axis=1, keepdims=True)
        out = jnp.sum(s * lw_ref[...], axis=0, keepdims=True)
        out_ref[...] = out * (_SCALE / (_N * _HID)) + lb_ref[...]


def kernel(x, edge_index, edge_weight, tc1a, cheb_a, tc2a, tc1b, cheb_b, tc2b,
           lin_w, lin_b):
    del edge_index, edge_weight
    (W1k0, W1k1, W1k2), B1 = _pack_taps_t(tc1a)
    W2, B2 = _pack_stacked_t(tc2a)
    W3, B3 = _pack_stacked_t(tc1b)
    W4, B4 = _pack_stacked_t(tc2b)
    Wa, ba = cheb_a
    Wb, bb = cheb_b
    Wa, Wb = Wa.T, Wb.T
    ba = ba.reshape(_HID, 1)
    bb = bb.reshape(_HID, 1)
    lb = lin_b.reshape(1, -1)

    nblocks = -(-_N // _BN)
    mask = (jnp.arange(nblocks * _BN, dtype=jnp.int32) < _N)
    mask = mask.astype(jnp.float32).reshape(nblocks, 1, _BN)

    full = lambda a: pl.BlockSpec(a.shape, lambda *_: tuple(0 for _ in a.shape))
    out = pl.pallas_call(
        functools.partial(_stgcn_block, nblocks=nblocks),
        grid=(nblocks,),
        in_specs=[
            pl.BlockSpec((_T, _BN, _F_IN), lambda i: (0, i, 0)),
            pl.BlockSpec((1, 1, _BN), lambda i: (i, 0, 0)),
            full(W1k0), full(W1k1), full(W1k2), full(B1), full(Wa), full(ba),
            full(W2), full(B2), full(W3), full(B3), full(Wb), full(bb),
            full(W4), full(B4), full(lin_w), full(lb),
        ],
        out_specs=pl.BlockSpec((1, lin_w.shape[1]), lambda i: (0, 0)),
        out_shape=jax.ShapeDtypeStruct((1, lin_w.shape[1]), jnp.float32),
        scratch_shapes=[pltpu.VMEM((1, 13 * _BN), jnp.float32)],
    )(x, mask, W1k0, W1k1, W1k2, B1, Wa, ba, W2, B2, W3, B3, Wb, bb, W4, B4,
      lin_w, lb)
    return out[0]
